# baseline (speedup 1.0000x reference)
import functools

import jax
import jax.numpy as jnp
from jax import lax
from jax.experimental import pallas as pl
from jax.experimental.pallas import tpu as pltpu
from jax.experimental.pallas import tpu_sc as plsc

_B, _S, _D, _P = 8, 2048, 128, 8
_LANES = 16
_NV = _D // _LANES
_WIN = 512
_DUMP = 32


def _sc_patch_pool(batch, lens_flat):
    mesh = plsc.VectorSubcoreMesh(core_axis_name="c", subcore_axis_name="s")

    @functools.partial(
        pl.kernel,
        out_type=jax.ShapeDtypeStruct((_B, _P, _D), jnp.float32),
        mesh=mesh,
        compiler_params=pltpu.CompilerParams(
            use_tc_tiling_on_sc=False, needs_layout_passes=False
        ),
        scratch_types=[
            pltpu.VMEM((_LANES,), jnp.int32),
            pltpu.VMEM((_WIN, _D), jnp.float32),
            pltpu.VMEM((_LANES, _D), jnp.float32),
            pltpu.VMEM((_LANES,), jnp.int32),
            pltpu.VMEM((2, _D), jnp.float32),
            pltpu.VMEM_SHARED((_DUMP + 1, _D), jnp.float32),
            pltpu.SemaphoreType.DMA,
            pltpu.SemaphoreType.DMA,
        ],
    )
    def k(batch_hbm, lens_hbm, out_hbm, lens_v, buf_v, pacc_v, idx_v,
          fin_v, shared_v, sem, osem):
        cid = lax.axis_index("c")
        sid = lax.axis_index("s")
        b = cid * 4 + sid // 4
        base = (sid % 4) * _WIN
        p0f = 2 * (sid % 4)
        r0 = (b % 4) * _P + p0f

        pltpu.async_copy(batch_hbm.at[b, pl.ds(base, _WIN)], buf_v, sem)
        start = jnp.minimum(8 * b, 64 - _LANES)
        pltpu.sync_copy(lens_hbm.at[pl.ds(start, _LANES)], lens_v)

        iota = lax.iota(jnp.int32, _LANES)
        zero_v = jnp.zeros((_LANES,), jnp.int32)
        raw = lens_v[...]
        cums = plsc.cumsum(raw)
        ofs = 8 * b - start
        base0 = jnp.where(
            ofs > 0, jnp.sum(jnp.where(iota == 7, cums, zero_v)), 0
        )
        to_v = cums - base0
        frm_v = to_v - raw
        toc_v = jnp.minimum(to_v, _S)
        frmc_v = jnp.minimum(frm_v, _S)
        n_v = toc_v - frmc_v
        lo_v = jnp.clip(frmc_v - base, 0, _WIN)
        hi_v = jnp.clip(toc_v - base, 0, _WIN)

        zf = jnp.zeros((_LANES,), jnp.float32)

        @pl.loop(8, _LANES)
        def _(r):
            for d0 in range(_NV):
                pacc_v[r, pl.ds(d0 * _LANES, _LANES)] = zf

        pltpu.sync_copy(pacc_v.at[pl.ds(8, 2)], shared_v.at[pl.ds(r0, 2)])
        idx_v[...] = jnp.where(iota < _P, (b % 4) * _P + iota, _DUMP)
        plsc.subcore_barrier()

        pltpu.make_async_copy(
            batch_hbm.at[b, pl.ds(base, _WIN)], buf_v, sem
        ).wait()

        @pl.loop(0, _P)
        def _(p):
            sel = iota == ofs + p
            lo = jnp.sum(jnp.where(sel, lo_v, zero_v))
            hi = jnp.sum(jnp.where(sel, hi_v, zero_v))
            cnt = hi - lo

            def body2(t, accs, lo=lo):
                j = lo + 2 * t
                return tuple(
                    a
                    + buf_v[j, pl.ds(d0 * _LANES, _LANES)]
                    + buf_v[j + 1, pl.ds(d0 * _LANES, _LANES)]
                    for d0, a in enumerate(accs)
                )

            accs = lax.fori_loop(0, cnt // 2, body2, (zf,) * _NV)
            jt = jnp.maximum(hi - 1, 0)
            odd = (cnt % 2) == 1
            for d0, a in enumerate(accs):
                tail = jnp.where(odd, buf_v[jt, pl.ds(d0 * _LANES, _LANES)], zf)
                pacc_v[p, pl.ds(d0 * _LANES, _LANES)] = a + tail

        pltpu.sync_copy(pacc_v, shared_v.at[idx_v], add=True)
        plsc.subcore_barrier()

        pltpu.sync_copy(shared_v.at[pl.ds(r0, 2)], fin_v)
        for i in range(2):
            p = p0f + i
            n = jnp.sum(jnp.where(iota == ofs + p, n_v, zero_v))
            denom = jnp.maximum(n, 1).astype(jnp.float32)
            empty = n == 0
            neg1 = jnp.full((_LANES,), -1.0, jnp.float32)
            for d0 in range(_NV):
                val = fin_v[i, pl.ds(d0 * _LANES, _LANES)] / denom
                fin_v[i, pl.ds(d0 * _LANES, _LANES)] = jnp.where(
                    empty, neg1, val
                )
            pltpu.async_copy(fin_v.at[i], out_hbm.at[b, p], osem)

        for i in range(2):
            pltpu.make_async_copy(
                fin_v.at[i], out_hbm.at[b, p0f + i], osem
            ).wait()

    return k(batch, lens_flat)


def kernel(batch, patch_lengths):
    return _sc_patch_pool(batch, jnp.reshape(patch_lengths, (_B * _P,)))
